# Initial kernel scaffold; baseline (speedup 1.0000x reference)
import jax
import jax.numpy as jnp
from jax.experimental import pallas as pl


def kernel(input_ids, word_table, pos_table, ln_gamma, ln_beta):
    raise NotImplementedError("write your pallas kernel here")


# speedup vs baseline: 1.3994x; 1.3994x over previous
import functools

import jax
import jax.numpy as jnp
from jax import lax
from jax.experimental import pallas as pl
from jax.experimental.pallas import tpu as pltpu
from jax.experimental.pallas import tpu_sc as plsc

_EPS = 1e-12

_NUM_CORES = 2
_NUM_SUBCORES = 16
_NUM_WORKERS = _NUM_CORES * _NUM_SUBCORES
_CHUNK = 128


def _sc_gather(word_table, flat_ids):
    n = flat_ids.shape[0]
    d = word_table.shape[1]
    b_per_w = n // _NUM_WORKERS
    n_chunks = b_per_w // _CHUNK
    mesh = plsc.VectorSubcoreMesh(core_axis_name="c", subcore_axis_name="s")

    @functools.partial(
        pl.kernel,
        mesh=mesh,
        out_type=jax.ShapeDtypeStruct((n, d), jnp.float32),
        scratch_types=[
            pltpu.VMEM((b_per_w,), jnp.int32),
            pltpu.VMEM((_CHUNK, d), jnp.float32),
            pltpu.SemaphoreType.DMA,
        ],
    )
    def gather_kernel(table_hbm, idx_hbm, out_hbm, idx_v, rows_v, sem):
        wid = lax.axis_index("s") * _NUM_CORES + lax.axis_index("c")
        base = wid * b_per_w
        pltpu.sync_copy(idx_hbm.at[pl.ds(base, b_per_w)], idx_v)

        @pl.loop(0, n_chunks)
        def _(c):
            off = pl.multiple_of(c * _CHUNK, _CHUNK)
            pltpu.async_copy(
                table_hbm.at[idx_v.at[pl.ds(off, _CHUNK)]],
                rows_v,
                sem,
            ).wait()
            pltpu.sync_copy(rows_v, out_hbm.at[pl.ds(base + off, _CHUNK)])

    return gather_kernel(word_table, flat_ids)


def _tc_add_ln(gathered, pos_table, ln_gamma, ln_beta):
    b, l, d = gathered.shape
    bb = 4

    def body(x_ref, pos_ref, g_ref, beta_ref, o_ref):
        x = x_ref[...] + pos_ref[...][None, :, :]
        mean = jnp.mean(x, axis=-1, keepdims=True)
        xc = x - mean
        var = jnp.mean(xc * xc, axis=-1, keepdims=True)
        inv = lax.rsqrt(var + _EPS)
        o_ref[...] = xc * inv * g_ref[...] + beta_ref[...]

    return pl.pallas_call(
        body,
        grid=(b // bb,),
        in_specs=[
            pl.BlockSpec((bb, l, d), lambda i: (i, 0, 0)),
            pl.BlockSpec((l, d), lambda i: (0, 0)),
            pl.BlockSpec((d,), lambda i: (0,)),
            pl.BlockSpec((d,), lambda i: (0,)),
        ],
        out_specs=pl.BlockSpec((bb, l, d), lambda i: (i, 0, 0)),
        out_shape=jax.ShapeDtypeStruct((b, l, d), jnp.float32),
    )(gathered, pos_table, ln_gamma, ln_beta)


def kernel(input_ids, word_table, pos_table, ln_gamma, ln_beta):
    b, l = input_ids.shape
    d = word_table.shape[1]
    flat_ids = input_ids.reshape(-1).astype(jnp.int32)
    gathered = _sc_gather(word_table, flat_ids)
    return _tc_add_ln(gathered.reshape(b, l, d), pos_table, ln_gamma, ln_beta)


# speedup vs baseline: 1.4332x; 1.0241x over previous
import functools

import jax
import jax.numpy as jnp
from jax import lax
from jax.experimental import pallas as pl
from jax.experimental.pallas import tpu as pltpu
from jax.experimental.pallas import tpu_sc as plsc

_EPS = 1e-12

_NUM_CORES = 2
_NUM_SUBCORES = 16
_NUM_WORKERS = _NUM_CORES * _NUM_SUBCORES
_CHUNK = 64


def _sc_gather(word_table, flat_ids):
    n = flat_ids.shape[0]
    d = word_table.shape[1]
    b_per_w = n // _NUM_WORKERS
    n_chunks = b_per_w // _CHUNK
    assert n_chunks % 2 == 0 and n_chunks >= 4
    mesh = plsc.VectorSubcoreMesh(core_axis_name="c", subcore_axis_name="s")

    @functools.partial(
        pl.kernel,
        mesh=mesh,
        out_type=jax.ShapeDtypeStruct((n, d), jnp.float32),
        scratch_types=[
            pltpu.VMEM((b_per_w,), jnp.int32),
            pltpu.VMEM((_CHUNK, d), jnp.float32),
            pltpu.VMEM((_CHUNK, d), jnp.float32),
            pltpu.SemaphoreType.DMA,
            pltpu.SemaphoreType.DMA,
        ],
    )
    def gather_kernel(table_hbm, idx_hbm, out_hbm, idx_v, rows0, rows1, sem0, sem1):
        wid = lax.axis_index("s") * _NUM_CORES + lax.axis_index("c")
        base = wid * b_per_w
        pltpu.sync_copy(idx_hbm.at[pl.ds(base, b_per_w)], idx_v)

        def gather_start(c, buf, sem):
            off = pl.multiple_of(c * _CHUNK, _CHUNK)
            return pltpu.make_async_copy(
                table_hbm.at[idx_v.at[pl.ds(off, _CHUNK)]], buf, sem
            )

        def write_out(c, buf):
            off = pl.multiple_of(c * _CHUNK, _CHUNK)
            pltpu.sync_copy(buf, out_hbm.at[pl.ds(base + off, _CHUNK)])

        gather_start(0, rows0, sem0).start()
        gather_start(1, rows1, sem1).start()

        @pl.loop(0, n_chunks - 2, step=2)
        def _(c):
            gather_start(c, rows0, sem0).wait()
            write_out(c, rows0)
            gather_start(c + 2, rows0, sem0).start()
            gather_start(c + 1, rows1, sem1).wait()
            write_out(c + 1, rows1)
            gather_start(c + 3, rows1, sem1).start()

        gather_start(n_chunks - 2, rows0, sem0).wait()
        write_out(n_chunks - 2, rows0)
        gather_start(n_chunks - 1, rows1, sem1).wait()
        write_out(n_chunks - 1, rows1)

    return gather_kernel(word_table, flat_ids)


def _tc_add_ln(gathered, pos_table, ln_gamma, ln_beta):
    b, l, d = gathered.shape
    bb = 4

    def body(x_ref, pos_ref, g_ref, beta_ref, o_ref):
        x = x_ref[...] + pos_ref[...][None, :, :]
        mean = jnp.mean(x, axis=-1, keepdims=True)
        xc = x - mean
        var = jnp.mean(xc * xc, axis=-1, keepdims=True)
        inv = lax.rsqrt(var + _EPS)
        o_ref[...] = xc * inv * g_ref[...] + beta_ref[...]

    return pl.pallas_call(
        body,
        grid=(b // bb,),
        in_specs=[
            pl.BlockSpec((bb, l, d), lambda i: (i, 0, 0)),
            pl.BlockSpec((l, d), lambda i: (0, 0)),
            pl.BlockSpec((d,), lambda i: (0,)),
            pl.BlockSpec((d,), lambda i: (0,)),
        ],
        out_specs=pl.BlockSpec((bb, l, d), lambda i: (i, 0, 0)),
        out_shape=jax.ShapeDtypeStruct((b, l, d), jnp.float32),
    )(gathered, pos_table, ln_gamma, ln_beta)


def kernel(input_ids, word_table, pos_table, ln_gamma, ln_beta):
    b, l = input_ids.shape
    d = word_table.shape[1]
    flat_ids = input_ids.reshape(-1).astype(jnp.int32)
    gathered = _sc_gather(word_table, flat_ids)
    return _tc_add_ln(gathered.reshape(b, l, d), pos_table, ln_gamma, ln_beta)
